# initial kernel scaffold (unmeasured)
import jax
import jax.numpy as jnp
from jax import lax
from jax.experimental import pallas as pl
from jax.experimental.pallas import tpu as pltpu


def kernel(
    x,
):
    def body(*refs):
        pass

    out_shape = jax.ShapeDtypeStruct(..., jnp.float32)
    return pl.pallas_call(body, out_shape=out_shape)(...)


# baseline (device time: 148596 ns/iter reference)
import jax
import jax.numpy as jnp
from jax import lax
from jax.experimental import pallas as pl
from jax.experimental.pallas import tpu as pltpu

N_DEV = 16


def kernel(x):
    m, n = x.shape
    m_per = m // N_DEV

    def body(
        x_ref,
        out_ref,
        send_buf,
        recv_buf,
        rs_send_sems,
        rs_recv_sems,
        ag_send_sems,
        ag_recv_sems,
    ):
        my = lax.axis_index("i")
        left = lax.rem(my + N_DEV - 1, N_DEV)
        right = lax.rem(my + 1, N_DEV)

        barrier_sem = pltpu.get_barrier_semaphore()
        for nbr in (left, right):
            pl.semaphore_signal(
                barrier_sem,
                inc=1,
                device_id=(nbr,),
                device_id_type=pl.DeviceIdType.MESH,
            )
        pl.semaphore_wait(barrier_sem, 2)

        for s in range(N_DEV - 1):
            c = lax.rem(my - s + N_DEV, N_DEV)
            chunk = x_ref[pl.ds(c * m_per, m_per), :].astype(jnp.bfloat16)
            if s == 0:
                acc = chunk
            else:
                acc = chunk + recv_buf[s - 1, :, :]
            send_buf[s, :, :] = acc
            rdma = pltpu.make_async_remote_copy(
                src_ref=send_buf.at[s],
                dst_ref=recv_buf.at[s],
                send_sem=rs_send_sems.at[s],
                recv_sem=rs_recv_sems.at[s],
                device_id=(right,),
                device_id_type=pl.DeviceIdType.MESH,
            )
            rdma.start()
            rdma.wait()

        owned = lax.rem(my + 1, N_DEV)
        final = (
            x_ref[pl.ds(owned * m_per, m_per), :].astype(jnp.bfloat16)
            + recv_buf[N_DEV - 2, :, :]
        )
        out_ref[pl.ds(owned * m_per, m_per), :] = final

        for h in range(N_DEV - 1):
            src_c = lax.rem(my + 1 - h + N_DEV, N_DEV)
            rows = pl.ds(src_c * m_per, m_per)
            rdma = pltpu.make_async_remote_copy(
                src_ref=out_ref.at[rows, :],
                dst_ref=out_ref.at[rows, :],
                send_sem=ag_send_sems.at[h],
                recv_sem=ag_recv_sems.at[h],
                device_id=(right,),
                device_id_type=pl.DeviceIdType.MESH,
            )
            rdma.start()
            rdma.wait()

    return pl.pallas_call(
        body,
        out_shape=jax.ShapeDtypeStruct((m, n), jnp.bfloat16),
        in_specs=[pl.BlockSpec(memory_space=pltpu.VMEM)],
        out_specs=pl.BlockSpec(memory_space=pltpu.VMEM),
        scratch_shapes=[
            pltpu.VMEM((N_DEV - 1, m_per, n), jnp.bfloat16),
            pltpu.VMEM((N_DEV - 1, m_per, n), jnp.bfloat16),
            pltpu.SemaphoreType.DMA((N_DEV - 1,)),
            pltpu.SemaphoreType.DMA((N_DEV - 1,)),
            pltpu.SemaphoreType.DMA((N_DEV - 1,)),
            pltpu.SemaphoreType.DMA((N_DEV - 1,)),
        ],
        compiler_params=pltpu.CompilerParams(collective_id=0),
    )(x)


# device time: 80991 ns/iter; 1.8347x vs baseline; 1.8347x over previous
import jax
import jax.numpy as jnp
from jax import lax
from jax.experimental import pallas as pl
from jax.experimental.pallas import tpu as pltpu

N_DEV = 16
MASKS = (1, 3, 4, 8)
ORDERS = ((1, 3, 4, 8), (4, 8, 1, 3))
N_STEPS = 4


def kernel(x):
    m, n = x.shape
    n_half = n // 2

    def body(
        x_ref,
        out_ref,
        recv_rs,
        rs_send_sems,
        rs_recv_sems,
        ag_send_sems,
        ag_recv_sems,
    ):
        my = lax.axis_index("i")
        coord = {
            1: (my ^ (my >> 1)) & 1,
            3: (my >> 1) & 1,
            4: (my >> 2) & 1,
            8: (my >> 3) & 1,
        }

        barrier_sem = pltpu.get_barrier_semaphore()
        for mask in MASKS:
            pl.semaphore_signal(
                barrier_sem,
                inc=1,
                device_id=(my ^ mask,),
                device_id_type=pl.DeviceIdType.MESH,
            )
        pl.semaphore_wait(barrier_sem, len(MASKS))

        out_ref[:, :] = x_ref[:, :].astype(jnp.bfloat16)

        lo = [jnp.int32(0), jnp.int32(0)]
        length = m
        for t in range(N_STEPS):
            half_rows = length // 2
            rdmas = []
            kept = []
            for h in (0, 1):
                mask = ORDERS[h][t]
                a = coord[mask]
                send_lo = lo[h] + (1 - a) * half_rows
                kept_lo = lo[h] + a * half_rows
                cols = pl.ds(h * n_half, n_half)
                rdma = pltpu.make_async_remote_copy(
                    src_ref=out_ref.at[pl.ds(send_lo, half_rows), cols],
                    dst_ref=recv_rs.at[h, t, pl.ds(0, half_rows), :],
                    send_sem=rs_send_sems.at[h, t],
                    recv_sem=rs_recv_sems.at[h, t],
                    device_id=(my ^ mask,),
                    device_id_type=pl.DeviceIdType.MESH,
                )
                rdma.start()
                rdmas.append(rdma)
                kept.append(kept_lo)
                lo[h] = kept_lo
            for h in (0, 1):
                rdmas[h].wait()
            for h in (0, 1):
                cols = pl.ds(h * n_half, n_half)
                rows = pl.ds(kept[h], half_rows)
                out_ref[rows, cols] = (
                    out_ref[rows, cols] + recv_rs[h, t, pl.ds(0, half_rows), :]
                )
            length = half_rows

        for t in reversed(range(N_STEPS)):
            rdmas = []
            for h in (0, 1):
                mask = ORDERS[h][t]
                a = coord[mask]
                cols = pl.ds(h * n_half, n_half)
                rows = pl.ds(lo[h], length)
                rdma = pltpu.make_async_remote_copy(
                    src_ref=out_ref.at[rows, cols],
                    dst_ref=out_ref.at[rows, cols],
                    send_sem=ag_send_sems.at[h, t],
                    recv_sem=ag_recv_sems.at[h, t],
                    device_id=(my ^ mask,),
                    device_id_type=pl.DeviceIdType.MESH,
                )
                rdma.start()
                rdmas.append(rdma)
                lo[h] = lo[h] - a * length
            for r in rdmas:
                r.wait()
            length *= 2

    return pl.pallas_call(
        body,
        out_shape=jax.ShapeDtypeStruct((m, n), jnp.bfloat16),
        in_specs=[pl.BlockSpec(memory_space=pltpu.VMEM)],
        out_specs=pl.BlockSpec(memory_space=pltpu.VMEM),
        scratch_shapes=[
            pltpu.VMEM((2, N_STEPS, m // 2, n_half), jnp.bfloat16),
            pltpu.SemaphoreType.DMA((2, N_STEPS)),
            pltpu.SemaphoreType.DMA((2, N_STEPS)),
            pltpu.SemaphoreType.DMA((2, N_STEPS)),
            pltpu.SemaphoreType.DMA((2, N_STEPS)),
        ],
        compiler_params=pltpu.CompilerParams(collective_id=0),
    )(x)


# device time: 67533 ns/iter; 2.2003x vs baseline; 1.1993x over previous
import jax
import jax.numpy as jnp
from jax import lax
from jax.experimental import pallas as pl
from jax.experimental.pallas import tpu as pltpu

N_DEV = 16
MASKS = (1, 3, 4, 8)
ORDERS = (
    (1, 3, 4, 8),
    (3, 1, 8, 4),
    (4, 8, 1, 3),
    (8, 4, 3, 1),
)
N_Q = 4
N_STEPS = 4


def kernel(x):
    m, n = x.shape
    n_q = n // N_Q

    def body(
        x_ref,
        out_ref,
        recv_rs,
        rs_send_sems,
        rs_recv_sems,
        ag_send_sems,
        ag_recv_sems,
    ):
        my = lax.axis_index("i")
        coord = {
            1: (my ^ (my >> 1)) & 1,
            3: (my >> 1) & 1,
            4: (my >> 2) & 1,
            8: (my >> 3) & 1,
        }

        barrier_sem = pltpu.get_barrier_semaphore()
        for mask in MASKS:
            pl.semaphore_signal(
                barrier_sem,
                inc=1,
                device_id=(my ^ mask,),
                device_id_type=pl.DeviceIdType.MESH,
            )
        pl.semaphore_wait(barrier_sem, len(MASKS))

        out_ref[:, :] = x_ref[:, :].astype(jnp.bfloat16)

        pending = []

        lo = [jnp.int32(0) for _ in range(N_Q)]
        length = m
        for t in range(N_STEPS):
            half_rows = length // 2
            rdmas = [None] * N_Q
            kept = [None] * N_Q
            for q in range(N_Q):
                mask = ORDERS[q][t]
                a = coord[mask]
                send_lo = lo[q] + (1 - a) * half_rows
                kept_lo = lo[q] + a * half_rows
                cols = pl.ds(q * n_q, n_q)
                rdma = pltpu.make_async_remote_copy(
                    src_ref=out_ref.at[pl.ds(send_lo, half_rows), cols],
                    dst_ref=recv_rs.at[q, t, pl.ds(0, half_rows), :],
                    send_sem=rs_send_sems.at[q, t],
                    recv_sem=rs_recv_sems.at[q, t],
                    device_id=(my ^ mask,),
                    device_id_type=pl.DeviceIdType.MESH,
                )
                rdma.start()
                rdmas[q] = rdma
                kept[q] = kept_lo
                lo[q] = kept_lo
            for mask in MASKS:
                q = next(i for i in range(N_Q) if ORDERS[i][t] == mask)
                rdmas[q].wait_recv()
                cols = pl.ds(q * n_q, n_q)
                rows = pl.ds(kept[q], half_rows)
                out_ref[rows, cols] = (
                    out_ref[rows, cols] + recv_rs[q, t, pl.ds(0, half_rows), :]
                )
            pending.extend(rdmas)
            length = half_rows

        for t in reversed(range(N_STEPS)):
            rdmas = [None] * N_Q
            for q in range(N_Q):
                mask = ORDERS[q][t]
                a = coord[mask]
                cols = pl.ds(q * n_q, n_q)
                rows = pl.ds(lo[q], length)
                rdma = pltpu.make_async_remote_copy(
                    src_ref=out_ref.at[rows, cols],
                    dst_ref=out_ref.at[rows, cols],
                    send_sem=ag_send_sems.at[q, t],
                    recv_sem=ag_recv_sems.at[q, t],
                    device_id=(my ^ mask,),
                    device_id_type=pl.DeviceIdType.MESH,
                )
                rdma.start()
                rdmas[q] = rdma
                lo[q] = lo[q] - a * length
            for q in range(N_Q):
                rdmas[q].wait_recv()
            pending.extend(rdmas)
            length *= 2

        for rdma in pending:
            rdma.wait_send()

    return pl.pallas_call(
        body,
        out_shape=jax.ShapeDtypeStruct((m, n), jnp.bfloat16),
        in_specs=[pl.BlockSpec(memory_space=pltpu.VMEM)],
        out_specs=pl.BlockSpec(memory_space=pltpu.VMEM),
        scratch_shapes=[
            pltpu.VMEM((N_Q, N_STEPS, m // 2, n_q), jnp.bfloat16),
            pltpu.SemaphoreType.DMA((N_Q, N_STEPS)),
            pltpu.SemaphoreType.DMA((N_Q, N_STEPS)),
            pltpu.SemaphoreType.DMA((N_Q, N_STEPS)),
            pltpu.SemaphoreType.DMA((N_Q, N_STEPS)),
        ],
        compiler_params=pltpu.CompilerParams(collective_id=0),
    )(x)


# device time: 62920 ns/iter; 2.3617x vs baseline; 1.0733x over previous
import jax
import jax.numpy as jnp
from jax import lax
from jax.experimental import pallas as pl
from jax.experimental.pallas import tpu as pltpu

N_DEV = 16
MASKS = (1, 3, 4, 8)
ORDERS = (
    (1, 3, 4, 8),
    (3, 1, 8, 4),
    (4, 8, 1, 3),
    (8, 4, 3, 1),
)
N_Q = 4
N_STEPS = 4
Q_FOR = tuple(
    {ORDERS[q][t]: q for q in range(N_Q)} for t in range(N_STEPS)
)


def kernel(x):
    m, n = x.shape
    n_q = n // N_Q

    def body(
        x_ref,
        out_ref,
        recv_rs,
        rs_send_sems,
        rs_recv_sems,
        ag_send_sems,
        ag_recv_sems,
    ):
        my = lax.axis_index("i")
        coord = {
            1: (my ^ (my >> 1)) & 1,
            3: (my >> 1) & 1,
            4: (my >> 2) & 1,
            8: (my >> 3) & 1,
        }
        A = [[coord[ORDERS[q][t]] for t in range(N_STEPS)] for q in range(N_Q)]

        def cols(q):
            return pl.ds(q * n_q, n_q)

        def rs_rdma(q, t, src_lo, rows):
            return pltpu.make_async_remote_copy(
                src_ref=out_ref.at[pl.ds(src_lo, rows), cols(q)],
                dst_ref=recv_rs.at[q, t, pl.ds(0, rows), :],
                send_sem=rs_send_sems.at[q, t],
                recv_sem=rs_recv_sems.at[q, t],
                device_id=(my ^ ORDERS[q][t],),
                device_id_type=pl.DeviceIdType.MESH,
            )

        def ag_rdma(q, t, src_lo, rows):
            rs = pl.ds(src_lo, rows)
            return pltpu.make_async_remote_copy(
                src_ref=out_ref.at[rs, cols(q)],
                dst_ref=out_ref.at[rs, cols(q)],
                send_sem=ag_send_sems.at[q, t],
                recv_sem=ag_recv_sems.at[q, t],
                device_id=(my ^ ORDERS[q][t],),
                device_id_type=pl.DeviceIdType.MESH,
            )

        barrier_sem = pltpu.get_barrier_semaphore()
        for mask in MASKS:
            pl.semaphore_signal(
                barrier_sem,
                inc=1,
                device_id=(my ^ mask,),
                device_id_type=pl.DeviceIdType.MESH,
            )
        pl.semaphore_wait(barrier_sem, len(MASKS))

        pending = []
        rs_inflight = [None] * N_Q
        ag_inflight = [None] * N_Q
        lo = [None] * N_Q

        half0 = m // 2
        for mask in MASKS:
            q = Q_FOR[0][mask]
            a = A[q][0]
            send_lo = (1 - a) * half0
            rows = pl.ds(send_lo, half0)
            out_ref[rows, cols(q)] = x_ref[rows, cols(q)].astype(jnp.bfloat16)
            r = rs_rdma(q, 0, send_lo, half0)
            r.start()
            rs_inflight[q] = r
        for q in range(N_Q):
            a = A[q][0]
            kept_lo = a * half0
            rows = pl.ds(kept_lo, half0)
            out_ref[rows, cols(q)] = x_ref[rows, cols(q)].astype(jnp.bfloat16)
            lo[q] = kept_lo

        for t in range(N_STEPS):
            rows_t = m >> (t + 1)
            for mask in MASKS:
                q = Q_FOR[t][mask]
                r = rs_inflight[q]
                r.wait_recv()
                pending.append(r)
                rr = pl.ds(lo[q], rows_t)
                out_ref[rr, cols(q)] = (
                    out_ref[rr, cols(q)] + recv_rs[q, t, pl.ds(0, rows_t), :]
                )
                if t < N_STEPS - 1:
                    hr = m >> (t + 2)
                    a = A[q][t + 1]
                    nxt = rs_rdma(q, t + 1, lo[q] + (1 - a) * hr, hr)
                    nxt.start()
                    rs_inflight[q] = nxt
                    lo[q] = lo[q] + a * hr
                else:
                    nxt = ag_rdma(q, N_STEPS - 1, lo[q], rows_t)
                    nxt.start()
                    ag_inflight[q] = nxt

        for t in reversed(range(N_STEPS)):
            rows_t = m >> (t + 1)
            for mask in MASKS:
                q = Q_FOR[t][mask]
                r = ag_inflight[q]
                r.wait_recv()
                pending.append(r)
                lo[q] = lo[q] - A[q][t] * rows_t
                if t > 0:
                    nxt = ag_rdma(q, t - 1, lo[q], m >> t)
                    nxt.start()
                    ag_inflight[q] = nxt

        for r in pending:
            r.wait_send()

    return pl.pallas_call(
        body,
        out_shape=jax.ShapeDtypeStruct((m, n), jnp.bfloat16),
        in_specs=[pl.BlockSpec(memory_space=pltpu.VMEM)],
        out_specs=pl.BlockSpec(memory_space=pltpu.VMEM),
        scratch_shapes=[
            pltpu.VMEM((N_Q, N_STEPS, m // 2, n_q), jnp.bfloat16),
            pltpu.SemaphoreType.DMA((N_Q, N_STEPS)),
            pltpu.SemaphoreType.DMA((N_Q, N_STEPS)),
            pltpu.SemaphoreType.DMA((N_Q, N_STEPS)),
            pltpu.SemaphoreType.DMA((N_Q, N_STEPS)),
        ],
        compiler_params=pltpu.CompilerParams(collective_id=0),
    )(x)


# device time: 56764 ns/iter; 2.6178x vs baseline; 1.1084x over previous
import jax
import jax.numpy as jnp
from jax import lax
from jax.experimental import pallas as pl
from jax.experimental.pallas import tpu as pltpu

N_DEV = 16
MASKS = (1, 3, 4, 8)
ORDERS = (
    (1, 3, 4, 8),
    (1, 3, 8, 4),
    (1, 4, 8, 3),
    (3, 1, 4, 8),
    (3, 1, 8, 4),
    (3, 8, 4, 1),
    (4, 1, 3, 8),
    (8, 3, 1, 4),
)
N_Q = 8
N_STEPS = 4
Q_FOR = tuple(
    {m: tuple(q for q in range(N_Q) if ORDERS[q][t] == m) for m in MASKS}
    for t in range(N_STEPS)
)


def kernel(x):
    m, n = x.shape
    n_q = n // N_Q

    def body(
        x_ref,
        out_ref,
        recv_rs,
        rs_send_sems,
        rs_recv_sems,
        ag_send_sems,
        ag_recv_sems,
    ):
        my = lax.axis_index("i")
        coord = {
            1: (my ^ (my >> 1)) & 1,
            3: (my >> 1) & 1,
            4: (my >> 2) & 1,
            8: (my >> 3) & 1,
        }
        A = [[coord[ORDERS[q][t]] for t in range(N_STEPS)] for q in range(N_Q)]

        def cols(q):
            return pl.ds(q * n_q, n_q)

        def rs_rdma(q, t, src_lo, rows):
            return pltpu.make_async_remote_copy(
                src_ref=out_ref.at[pl.ds(src_lo, rows), cols(q)],
                dst_ref=recv_rs.at[q, t, pl.ds(0, rows), :],
                send_sem=rs_send_sems.at[q, t],
                recv_sem=rs_recv_sems.at[q, t],
                device_id=(my ^ ORDERS[q][t],),
                device_id_type=pl.DeviceIdType.MESH,
            )

        def ag_rdma(q, t, src_lo, rows):
            rs = pl.ds(src_lo, rows)
            return pltpu.make_async_remote_copy(
                src_ref=out_ref.at[rs, cols(q)],
                dst_ref=out_ref.at[rs, cols(q)],
                send_sem=ag_send_sems.at[q, t],
                recv_sem=ag_recv_sems.at[q, t],
                device_id=(my ^ ORDERS[q][t],),
                device_id_type=pl.DeviceIdType.MESH,
            )

        barrier_sem = pltpu.get_barrier_semaphore()
        for mask in MASKS:
            pl.semaphore_signal(
                barrier_sem,
                inc=1,
                device_id=(my ^ mask,),
                device_id_type=pl.DeviceIdType.MESH,
            )
        pl.semaphore_wait(barrier_sem, len(MASKS))

        pending = []
        rs_inflight = [None] * N_Q
        ag_inflight = [None] * N_Q
        lo = [None] * N_Q

        half0 = m // 2
        for mask in reversed(MASKS):
            for q in Q_FOR[0][mask]:
                a = A[q][0]
                send_lo = (1 - a) * half0
                rows = pl.ds(send_lo, half0)
                out_ref[rows, cols(q)] = x_ref[rows, cols(q)].astype(
                    jnp.bfloat16
                )
                r = rs_rdma(q, 0, send_lo, half0)
                r.start()
                rs_inflight[q] = r
        for q in range(N_Q):
            a = A[q][0]
            kept_lo = a * half0
            rows = pl.ds(kept_lo, half0)
            out_ref[rows, cols(q)] = x_ref[rows, cols(q)].astype(jnp.bfloat16)
            lo[q] = kept_lo

        for t in range(N_STEPS):
            rows_t = m >> (t + 1)
            for mask in MASKS:
                for q in Q_FOR[t][mask]:
                    r = rs_inflight[q]
                    r.wait_recv()
                    pending.append(r)
                    rr = pl.ds(lo[q], rows_t)
                    out_ref[rr, cols(q)] = (
                        out_ref[rr, cols(q)]
                        + recv_rs[q, t, pl.ds(0, rows_t), :]
                    )
                    if t < N_STEPS - 1:
                        hr = m >> (t + 2)
                        a = A[q][t + 1]
                        nxt = rs_rdma(q, t + 1, lo[q] + (1 - a) * hr, hr)
                        nxt.start()
                        rs_inflight[q] = nxt
                        lo[q] = lo[q] + a * hr
                    else:
                        nxt = ag_rdma(q, N_STEPS - 1, lo[q], rows_t)
                        nxt.start()
                        ag_inflight[q] = nxt

        for t in reversed(range(N_STEPS)):
            rows_t = m >> (t + 1)
            for mask in MASKS:
                for q in Q_FOR[t][mask]:
                    r = ag_inflight[q]
                    r.wait_recv()
                    pending.append(r)
                    lo[q] = lo[q] - A[q][t] * rows_t
                    if t > 0:
                        nxt = ag_rdma(q, t - 1, lo[q], m >> t)
                        nxt.start()
                        ag_inflight[q] = nxt

        for r in pending:
            r.wait_send()

    return pl.pallas_call(
        body,
        out_shape=jax.ShapeDtypeStruct((m, n), jnp.bfloat16),
        in_specs=[pl.BlockSpec(memory_space=pltpu.VMEM)],
        out_specs=pl.BlockSpec(memory_space=pltpu.VMEM),
        scratch_shapes=[
            pltpu.VMEM((N_Q, N_STEPS, m // 2, n_q), jnp.bfloat16),
            pltpu.SemaphoreType.DMA((N_Q, N_STEPS)),
            pltpu.SemaphoreType.DMA((N_Q, N_STEPS)),
            pltpu.SemaphoreType.DMA((N_Q, N_STEPS)),
            pltpu.SemaphoreType.DMA((N_Q, N_STEPS)),
        ],
        compiler_params=pltpu.CompilerParams(collective_id=0),
    )(x)


# device time: 53869 ns/iter; 2.7585x vs baseline; 1.0537x over previous
import jax
import jax.numpy as jnp
from jax import lax
from jax.experimental import pallas as pl
from jax.experimental.pallas import tpu as pltpu

N_DEV = 16
MASKS = (1, 3, 4, 8)
ORDERS = (
    (1, 3, 4, 8),
    (1, 3, 8, 4),
    (1, 4, 8, 3),
    (3, 1, 4, 8),
    (3, 1, 8, 4),
    (3, 8, 4, 1),
    (4, 1, 3, 8),
    (8, 3, 1, 4),
)
N_Q = 8
N_STEPS = 4
Q_FOR = tuple(
    {m: tuple(q for q in range(N_Q) if ORDERS[q][t] == m) for m in MASKS}
    for t in range(N_STEPS)
)


def _lsb(v: int) -> int:
    return (v & -v).bit_length() - 1


M_ORDER = [0]
for _t in (3, 2, 1, 0):
    M_ORDER.extend(r | (1 << _t) for r in list(M_ORDER))


def kernel(x):
    m, n = x.shape
    n_q = n // N_Q
    chunk_rows = m // N_DEV

    def body(
        x_ref,
        out_ref,
        recv_rs,
        rs_send_sems,
        rs_recv_sems,
        ag_send_sems,
        ag_recv_sems,
    ):
        my = lax.axis_index("i")
        coord = {
            1: (my ^ (my >> 1)) & 1,
            3: (my >> 1) & 1,
            4: (my >> 2) & 1,
            8: (my >> 3) & 1,
        }
        A = [[coord[ORDERS[q][t]] for t in range(N_STEPS)] for q in range(N_Q)]

        def cols(q):
            return pl.ds(q * n_q, n_q)

        def chunk_lo(q, r):
            return sum(
                (A[q][t] ^ ((r >> t) & 1)) * (m >> (t + 1))
                for t in range(N_STEPS)
            )

        def rs_rdma(q, t, src_lo, rows):
            return pltpu.make_async_remote_copy(
                src_ref=out_ref.at[pl.ds(src_lo, rows), cols(q)],
                dst_ref=recv_rs.at[q, t, pl.ds(0, rows), :],
                send_sem=rs_send_sems.at[q, t],
                recv_sem=rs_recv_sems.at[q, t],
                device_id=(my ^ ORDERS[q][t],),
                device_id_type=pl.DeviceIdType.MESH,
            )

        def ag_rdma(q, r, t):
            u = r | (1 << t)
            rr = pl.ds(chunk_lo(q, r), chunk_rows)
            return pltpu.make_async_remote_copy(
                src_ref=out_ref.at[rr, cols(q)],
                dst_ref=out_ref.at[rr, cols(q)],
                send_sem=ag_send_sems.at[q, u],
                recv_sem=ag_recv_sems.at[q, u],
                device_id=(my ^ ORDERS[q][t],),
                device_id_type=pl.DeviceIdType.MESH,
            )

        def ag_wait_rdma(q, v):
            rr = pl.ds(chunk_lo(q, v), chunk_rows)
            return pltpu.make_async_remote_copy(
                src_ref=out_ref.at[rr, cols(q)],
                dst_ref=out_ref.at[rr, cols(q)],
                send_sem=ag_send_sems.at[q, v],
                recv_sem=ag_recv_sems.at[q, v],
                device_id=(my ^ ORDERS[q][_lsb(v)],),
                device_id_type=pl.DeviceIdType.MESH,
            )

        barrier_sem = pltpu.get_barrier_semaphore()
        for mask in MASKS:
            pl.semaphore_signal(
                barrier_sem,
                inc=1,
                device_id=(my ^ mask,),
                device_id_type=pl.DeviceIdType.MESH,
            )
        pl.semaphore_wait(barrier_sem, len(MASKS))

        pending = []
        rs_inflight = [None] * N_Q
        lo = [None] * N_Q

        half0 = m // 2
        for mask in reversed(MASKS):
            for q in Q_FOR[0][mask]:
                a = A[q][0]
                send_lo = (1 - a) * half0
                rows = pl.ds(send_lo, half0)
                out_ref[rows, cols(q)] = x_ref[rows, cols(q)].astype(
                    jnp.bfloat16
                )
                r = rs_rdma(q, 0, send_lo, half0)
                r.start()
                rs_inflight[q] = r
        for q in range(N_Q):
            a = A[q][0]
            kept_lo = a * half0
            rows = pl.ds(kept_lo, half0)
            out_ref[rows, cols(q)] = x_ref[rows, cols(q)].astype(jnp.bfloat16)
            lo[q] = kept_lo

        for t in range(N_STEPS):
            rows_t = m >> (t + 1)
            for mask in MASKS:
                for q in Q_FOR[t][mask]:
                    r = rs_inflight[q]
                    r.wait_recv()
                    pending.append(r)
                    rr = pl.ds(lo[q], rows_t)
                    out_ref[rr, cols(q)] = (
                        out_ref[rr, cols(q)]
                        + recv_rs[q, t, pl.ds(0, rows_t), :]
                    )
                    if t < N_STEPS - 1:
                        hr = m >> (t + 2)
                        a = A[q][t + 1]
                        nxt = rs_rdma(q, t + 1, lo[q] + (1 - a) * hr, hr)
                        nxt.start()
                        rs_inflight[q] = nxt
                        lo[q] = lo[q] + a * hr
                    else:
                        for td in reversed(range(N_STEPS)):
                            snd = ag_rdma(q, 0, td)
                            snd.start()
                            pending.append(snd)

        for v in M_ORDER[1:]:
            t_arr = _lsb(v)
            for mask in MASKS:
                for q in Q_FOR[t_arr][mask]:
                    ag_wait_rdma(q, v).wait_recv()
                    for td in reversed(range(t_arr)):
                        snd = ag_rdma(q, v, td)
                        snd.start()
                        pending.append(snd)

        for r in pending:
            r.wait_send()

    return pl.pallas_call(
        body,
        out_shape=jax.ShapeDtypeStruct((m, n), jnp.bfloat16),
        in_specs=[pl.BlockSpec(memory_space=pltpu.VMEM)],
        out_specs=pl.BlockSpec(memory_space=pltpu.VMEM),
        scratch_shapes=[
            pltpu.VMEM((N_Q, N_STEPS, m // 2, n_q), jnp.bfloat16),
            pltpu.SemaphoreType.DMA((N_Q, N_STEPS)),
            pltpu.SemaphoreType.DMA((N_Q, N_STEPS)),
            pltpu.SemaphoreType.DMA((N_Q, N_DEV)),
            pltpu.SemaphoreType.DMA((N_Q, N_DEV)),
        ],
        compiler_params=pltpu.CompilerParams(collective_id=0),
    )(x)
